# baseline (speedup 1.0000x reference)
import functools

import jax
import jax.numpy as jnp
from jax import lax
from jax.experimental import pallas as pl
from jax.experimental.pallas import tpu as pltpu
from jax.experimental.pallas import tpu_sc as plsc

T = 32768
V = 1024
BLANK = V - 1


NC = 2
NS = 16
NW = NC * NS

S_SC = T
RW = S_SC // NW
CH = 32
NCH = RW // CH
UNR = 8
VP = V + 1

_mesh = plsc.VectorSubcoreMesh(core_axis_name="c", subcore_axis_name="s")


@functools.partial(
    pl.kernel,
    mesh=_mesh,
    compiler_params=pltpu.CompilerParams(
        use_tc_tiling_on_sc=False, needs_layout_passes=False),
    out_type=[
        jax.ShapeDtypeStruct((S_SC,), jnp.int32),
        jax.ShapeDtypeStruct((S_SC,), jnp.int32),
        jax.ShapeDtypeStruct((S_SC,), jnp.float32),
    ],
    scratch_types=[
        pltpu.VMEM((2, CH, VP), jnp.float32),
        pltpu.VMEM((RW,), jnp.int32),
        pltpu.VMEM((RW,), jnp.float32),
        pltpu.VMEM((RW,), jnp.int32),
        pltpu.VMEM((V,), jnp.float32),
        pltpu.VMEM((16,), jnp.float32),
        pltpu.VMEM((16,), jnp.int32),
        pltpu.SemaphoreType.DMA,
        pltpu.SemaphoreType.DMA,
    ],
)
def _sc_decode(em, idx_hbm, keep_hbm, score_hbm,
               buf, idxb, scoreb, keepb, rowb, hvb, hpb, sem0, sem1):
    wid = lax.axis_index("c") * NS + lax.axis_index("s")
    start = wid * RW
    iota = lax.broadcasted_iota(jnp.int32, (16,), 0)

    @pl.when(wid > 0)
    def _():
        pltpu.sync_copy(em.at[start - 1], rowb)

    def _strip(s, carry):
        bv, bp = carry
        v = rowb[pl.ds(s * 16, 16)]
        gt = v > bv
        return jnp.where(gt, v, bv), jnp.where(gt, s * 16 + iota, bp)

    hv, hp = lax.fori_loop(1, V // 16, _strip, (rowb[pl.ds(0, 16)], iota))
    hm = hv[0]
    hpos = hp[0]
    for l in range(1, 16):
        vl = hv[l]
        pl_ = hp[l]
        gt = vl > hm
        eq = vl == hm
        hpos = jnp.where(gt, pl_, jnp.where(eq, jnp.minimum(hpos, pl_), hpos))
        hm = jnp.maximum(hm, vl)
    prev0 = jnp.where(wid > 0, hpos, -1)

    sems = (sem0, sem1)
    pltpu.async_copy(em.at[pl.ds(start, CH)], buf.at[0, :, pl.ds(0, V)], sem0)
    ninf = jnp.full((16,), -jnp.inf, jnp.float32)
    zero16 = jnp.zeros((16,), jnp.int32)

    for k in range(NCH):
        slot = k % 2
        if k + 1 < NCH:
            pltpu.async_copy(
                em.at[pl.ds(start + (k + 1) * CH, CH)],
                buf.at[1 - slot, :, pl.ds(0, V)], sems[1 - slot])
        pltpu.make_async_copy(
            em.at[pl.ds(start + k * CH, CH)],
            buf.at[slot, :, pl.ds(0, V)], sems[slot]).wait()
        bslot = buf.at[slot]
        for g in range(CH // 16):
            rowsel = iota + g * 16

            def _col(j, carry, rowsel=rowsel, bslot=bslot):
                bv, bp = carry
                c0 = jnp.full((16,), UNR * j, jnp.int32)
                cols = [c0 + c if c else c0 for c in range(UNR)]
                pairs = [
                    (plsc.load_gather(bslot, [rowsel, cc]), cc) for cc in cols
                ]
                while len(pairs) > 1:
                    nxt = []
                    for a in range(0, len(pairs), 2):
                        (va, pa), (vb, pb) = pairs[a], pairs[a + 1]
                        ab = va >= vb
                        nxt.append(
                            (jnp.maximum(va, vb), jnp.where(ab, pa, pb)))
                    pairs = nxt
                vm, pm = pairs[0]
                gt = vm > bv
                return jnp.where(gt, vm, bv), jnp.where(gt, pm, bp)

            bv, bp = lax.fori_loop(0, V // UNR, _col, (ninf, zero16))
            base = k * CH + g * 16
            idxb[pl.ds(base, 16)] = bp
            scoreb[pl.ds(base, 16)] = bv

    for q in range(RW // 16):
        cur = idxb[pl.ds(q * 16, 16)]
        if q == 0:
            pv = plsc.load_gather(idxb, [jnp.maximum(iota - 1, 0)])
            pv = jnp.where(iota == 0, prev0, pv)
        else:
            pv = plsc.load_gather(idxb, [q * 16 - 1 + iota])
        kp = (cur != BLANK) & (cur != pv)
        keepb[pl.ds(q * 16, 16)] = kp.astype(jnp.int32)

    pltpu.sync_copy(idxb, idx_hbm.at[pl.ds(start, RW)])
    pltpu.sync_copy(keepb, keep_hbm.at[pl.ds(start, RW)])
    pltpu.sync_copy(scoreb, score_hbm.at[pl.ds(start, RW)])


BT = 2048
NBLK = T // BT


def _tc_body(x_ref, idx_ref, keep_ref, score_ref, prev_ref):
    i = pl.program_id(0)

    @pl.when(i == 0)
    def _():
        prev_ref[0] = -1

    v = x_ref[:, 0:128]
    bestc = jnp.zeros((BT, 128), jnp.int32)
    for c in range(1, 8):
        u = x_ref[:, c * 128 : (c + 1) * 128]
        gt = u > v
        v = jnp.where(gt, u, v)
        bestc = jnp.where(gt, c, bestc)
    vT = v.T
    cT = bestc.T
    m = jnp.max(vT, axis=0)
    lane0 = jax.lax.broadcasted_iota(jnp.int32, (128, BT), 0)
    posT = cT * 128 + lane0
    cand = jnp.where(vT == m[None, :], posT, V)
    idx = jnp.min(cand, axis=0)
    prev_first = jnp.full((1,), prev_ref[0], dtype=jnp.int32)
    prev = jnp.concatenate([prev_first, idx[: BT - 1]])
    keep = (idx != BLANK) & (idx != prev)
    idx_ref[...] = idx
    keep_ref[...] = keep
    score_ref[...] = m
    prev_ref[0] = idx[BT - 1]


def _tc_decode(emission):
    return pl.pallas_call(
        _tc_body,
        grid=(NBLK,),
        in_specs=[pl.BlockSpec((BT, V), lambda i: (i, 0))],
        out_specs=[
            pl.BlockSpec((BT,), lambda i: (i,)),
            pl.BlockSpec((BT,), lambda i: (i,)),
            pl.BlockSpec((BT,), lambda i: (i,)),
        ],
        out_shape=[
            jax.ShapeDtypeStruct((T,), jnp.int32),
            jax.ShapeDtypeStruct((T,), jnp.bool_),
            jax.ShapeDtypeStruct((T,), jnp.float32),
        ],
        scratch_shapes=[pltpu.SMEM((1,), jnp.int32)],
    )(emission)


def kernel(emission):
    idx, keep_i, scores = _sc_decode(emission)
    return idx, keep_i.astype(bool), scores


# speedup vs baseline: 4.3479x; 4.3479x over previous
import jax
import jax.numpy as jnp
from jax.experimental import pallas as pl
from jax.experimental.pallas import tpu as pltpu

T = 32768
V = 1024
BLANK = V - 1
BT = 2048
NBLK = T // BT


def _body(x_ref, idx_ref, keep_ref, score_ref, prev_ref):
    i = pl.program_id(0)

    @pl.when(i == 0)
    def _():
        prev_ref[0] = -1

    v = x_ref[:, 0:128]
    bestc = jnp.zeros((BT, 128), jnp.int32)
    for c in range(1, 8):
        u = x_ref[:, c * 128 : (c + 1) * 128]
        gt = u > v
        v = jnp.where(gt, u, v)
        bestc = jnp.where(gt, c, bestc)
    vT = v.T
    cT = bestc.T
    m = jnp.max(vT, axis=0)
    lane0 = jax.lax.broadcasted_iota(jnp.int32, (128, BT), 0)
    posT = cT * 128 + lane0
    cand = jnp.where(vT == m[None, :], posT, V)
    idx = jnp.min(cand, axis=0)
    prev_first = jnp.full((1,), prev_ref[0], dtype=jnp.int32)
    prev = jnp.concatenate([prev_first, idx[: BT - 1]])
    keep = (idx != BLANK) & (idx != prev)
    idx_ref[...] = idx
    keep_ref[...] = keep
    score_ref[...] = m
    prev_ref[0] = idx[BT - 1]


def kernel(emission):
    idx, keep, scores = pl.pallas_call(
        _body,
        grid=(NBLK,),
        in_specs=[pl.BlockSpec((BT, V), lambda i: (i, 0))],
        out_specs=[
            pl.BlockSpec((BT,), lambda i: (i,)),
            pl.BlockSpec((BT,), lambda i: (i,)),
            pl.BlockSpec((BT,), lambda i: (i,)),
        ],
        out_shape=[
            jax.ShapeDtypeStruct((T,), jnp.int32),
            jax.ShapeDtypeStruct((T,), jnp.bool_),
            jax.ShapeDtypeStruct((T,), jnp.float32),
        ],
        scratch_shapes=[pltpu.SMEM((1,), jnp.int32)],
    )(emission)
    return idx, keep, scores
